# baseline (speedup 1.0000x reference)
import jax, jax.numpy as jnp
from jax import lax
from jax.experimental import pallas as pl
from jax.experimental.pallas import tpu as pltpu, tpu_sc as plsc

NC, NS = 2, 16
NW = NC * NS
CHUNK = 80
NBUF = 2
GROUPS = CHUNK // 16


def body(ids_hbm, table_hbm, out_hbm, idx_all, tab_v, rows0, rows1, ssem0, ssem1):
    rows = (rows0, rows1)
    ssem = (ssem0, ssem1)
    wid = lax.axis_index("s") * NC + lax.axis_index("c")
    n = ids_hbm.shape[0]
    pw = n // NW
    base = wid * pw
    nch = pw // CHUNK
    dim = tab_v.shape[1]

    pltpu.sync_copy(ids_hbm.at[pl.ds(base, pw)], idx_all)
    pltpu.sync_copy(table_hbm, tab_v)

    lane = lax.iota(jnp.int32, 16)

    def wait_scatter(b):
        pltpu.make_async_copy(rows[b], out_hbm.at[pl.ds(0, CHUNK)], ssem[b]).wait()

    def expand_chunk(ci, b):
        for g in range(GROUPS):
            toff = g * 16
            ids_v = idx_all[pl.ds(ci * CHUNK + toff, 16)]
            rowpos = lane + toff

            @pl.loop(0, dim, unroll=8)
            def _cols(j):
                col = jnp.zeros((16,), jnp.int32) + j
                v = plsc.load_gather(tab_v, [ids_v, col])
                plsc.store_scatter(rows[b], [rowpos, col], v)

    @pl.loop(0, nch, step=NBUF)
    def _pipe(i):
        for b in range(NBUF):
            ci = i + b

            @pl.when(ci >= NBUF)
            def _w():
                wait_scatter(b)

            expand_chunk(ci, b)
            pltpu.async_copy(rows[b], out_hbm.at[pl.ds(base + ci * CHUNK, CHUNK)], ssem[b])

    for b in range(NBUF):
        wait_scatter(b)


def kernel(token_ids, table):
    b, s = token_ids.shape
    v, d = table.shape
    ids = token_ids.reshape(-1).astype(jnp.int32)
    n = ids.shape[0]
    mesh = plsc.VectorSubcoreMesh(core_axis_name="c", subcore_axis_name="s",
                                  num_cores=NC, num_subcores=NS)
    out = pl.kernel(
        body, out_type=jax.ShapeDtypeStruct((n, d), jnp.float32), mesh=mesh,
        compiler_params=pltpu.CompilerParams(needs_layout_passes=False),
        scratch_types=[
            pltpu.VMEM((n // NW,), jnp.int32),
            pltpu.VMEM((v, d), jnp.float32),
            pltpu.VMEM((CHUNK, d), jnp.float32),
            pltpu.VMEM((CHUNK, d), jnp.float32),
            pltpu.SemaphoreType.DMA,
            pltpu.SemaphoreType.DMA,
        ],
    )(ids, table)
    return out.reshape(b, s, d)


# speedup vs baseline: 23.7762x; 23.7762x over previous
import jax, jax.numpy as jnp
from jax import lax
from jax.experimental import pallas as pl
from jax.experimental.pallas import tpu as pltpu, tpu_sc as plsc

NC, NS = 2, 16
NW = NC * NS
DEPTH = 4


def body(ids_hbm, table_hbm, out_hbm, idx_all, tab_v, sem):
    wid = lax.axis_index("s") * NC + lax.axis_index("c")
    n = ids_hbm.shape[0]
    pw = n // NW
    base = wid * pw
    ngroups = pw // 16

    pltpu.sync_copy(ids_hbm.at[pl.ds(base, pw)], idx_all)
    pltpu.sync_copy(table_hbm, tab_v)

    def issue_group(g):
        ids_v = idx_all[pl.ds(g * 16, 16)]
        goff = base + g * 16
        for l in range(16):
            tid = ids_v[l]
            pltpu.async_copy(tab_v.at[pl.ds(tid, 1)],
                             out_hbm.at[pl.ds(goff + l, 1)], sem)

    def drain_group():
        pltpu.make_async_copy(tab_v.at[pl.ds(0, 16)],
                              out_hbm.at[pl.ds(0, 16)], sem).wait()

    @pl.loop(0, DEPTH)
    def _prime(g):
        issue_group(g)

    @pl.loop(DEPTH, ngroups)
    def _steady(g):
        drain_group()
        issue_group(g)

    @pl.loop(0, DEPTH)
    def _tail(g):
        drain_group()


def kernel(token_ids, table):
    b, s = token_ids.shape
    v, d = table.shape
    ids = token_ids.reshape(-1).astype(jnp.int32)
    n = ids.shape[0]
    mesh = plsc.VectorSubcoreMesh(core_axis_name="c", subcore_axis_name="s",
                                  num_cores=NC, num_subcores=NS)
    out = pl.kernel(
        body, out_type=jax.ShapeDtypeStruct((n, d), jnp.float32), mesh=mesh,
        compiler_params=pltpu.CompilerParams(needs_layout_passes=False),
        scratch_types=[
            pltpu.VMEM((n // NW,), jnp.int32),
            pltpu.VMEM((v, d), jnp.float32),
            pltpu.SemaphoreType.DMA,
        ],
    )(ids, table)
    return out.reshape(b, s, d)
